# baseline (speedup 1.0000x reference)
import functools
import math

import numpy as np
import jax
import jax.numpy as jnp
from jax import lax
from jax.experimental import pallas as pl
from jax.experimental.pallas import tpu as pltpu
from jax.experimental.pallas import tpu_sc as plsc

_EMB = 128
_SEQ = 4096


@functools.lru_cache(maxsize=None)
def _positional_encoding(seq, emb):
    positions = np.arange(0, seq, dtype=np.float32)[:, None]
    div_term = 10000.0 ** (np.arange(0, emb, 2, dtype=np.float32) / emb)
    pe = np.zeros((seq, emb), dtype=np.float32)
    pe[:, 0::2] = np.sin(positions / div_term)
    pe[:, 1::2] = np.cos(positions / div_term)
    return pe


def kernel(x, table):
    b, s = x.shape
    v, d = table.shape
    n = b * s
    scale = float(math.sqrt(d))

    info = plsc.get_sparse_core_info()
    nc, ns, lanes = info.num_cores, info.num_subcores, info.num_lanes
    nw = nc * ns
    blk = s // nw
    assert blk <= 128
    nsub = 1
    sub = blk // nsub

    pe = jnp.asarray(_positional_encoding(s, d))

    mesh = plsc.VectorSubcoreMesh(core_axis_name="c", subcore_axis_name="s")

    @functools.partial(
        pl.kernel,
        mesh=mesh,
        out_type=jax.ShapeDtypeStruct((n, d), jnp.float32),
        scratch_types=[
            pltpu.VMEM((b, blk), jnp.int32),
            pltpu.VMEM((blk, d), jnp.float32),
        ]
        + [pltpu.VMEM((blk, d), jnp.float32) for _ in range(b)]
        + [
            pltpu.SemaphoreType.DMA,
            pltpu.SemaphoreType.DMA,
            pltpu.SemaphoreType.DMA,
        ],
    )
    def emb_kernel(idx_hbm, table_hbm, pe_hbm, out_hbm, idx_v, pe_v, *rest):
        rows = rest[:b]
        gsem, psem, osem = rest[b:]
        wid = lax.axis_index("s") * nc + lax.axis_index("c")
        pos0 = wid * blk

        pltpu.sync_copy(idx_hbm.at[:, pl.ds(pos0, blk)], idx_v)
        pe_cp = pltpu.async_copy(pe_hbm.at[pl.ds(pos0, blk)], pe_v, psem)
        gathers = []
        for r in range(nsub):
            for j in range(b):
                gathers.append(
                    pltpu.async_copy(
                        table_hbm.at[idx_v.at[j, pl.ds(r * sub, sub)]],
                        rows[j].at[pl.ds(r * sub, sub)],
                        gsem,
                    )
                )
        pe_cp.wait()

        stores = []
        for r in range(nsub):
            for j in range(b):
                gathers[r * b + j].wait()

            def body(i, _):
                for k in range(d // lanes):
                    sl = pl.ds(k * lanes, lanes)
                    pv = pe_v[i, sl]
                    for j in range(b):
                        rows[j][i, sl] = rows[j][i, sl] * scale + pv
                return 0

            lax.fori_loop(r * sub, (r + 1) * sub, body, 0)
            for j in range(b):
                stores.append(
                    pltpu.async_copy(
                        rows[j].at[pl.ds(r * sub, sub)],
                        out_hbm.at[pl.ds(j * s + pos0 + r * sub, sub)],
                        osem,
                    )
                )
        for st in stores:
            st.wait()

    out = emb_kernel(x, table, pe)
    return out.reshape(b, s, d)


# speedup vs baseline: 1.0962x; 1.0962x over previous
import functools
import math

import numpy as np
import jax
import jax.numpy as jnp
from jax import lax
from jax.experimental import pallas as pl
from jax.experimental.pallas import tpu as pltpu
from jax.experimental.pallas import tpu_sc as plsc

_EMB = 128
_SEQ = 4096


@functools.lru_cache(maxsize=None)
def _positional_encoding(seq, emb):
    positions = np.arange(0, seq, dtype=np.float32)[:, None]
    div_term = 10000.0 ** (np.arange(0, emb, 2, dtype=np.float32) / emb)
    pe = np.zeros((seq, emb), dtype=np.float32)
    pe[:, 0::2] = np.sin(positions / div_term)
    pe[:, 1::2] = np.cos(positions / div_term)
    return pe


def kernel(x, table):
    b, s = x.shape
    v, d = table.shape
    n = b * s
    scale = float(math.sqrt(d))

    info = plsc.get_sparse_core_info()
    nc, ns, lanes = info.num_cores, info.num_subcores, info.num_lanes
    nw = nc * ns
    blk = s // nw
    assert blk <= 128
    nsub = 2
    sub = blk // nsub

    pe = jnp.asarray(_positional_encoding(s, d))

    mesh = plsc.VectorSubcoreMesh(core_axis_name="c", subcore_axis_name="s")

    @functools.partial(
        pl.kernel,
        mesh=mesh,
        out_type=jax.ShapeDtypeStruct((n, d), jnp.float32),
        scratch_types=[
            pltpu.VMEM((b, blk), jnp.int32),
            pltpu.VMEM((blk, d), jnp.float32),
        ]
        + [pltpu.VMEM((blk, d), jnp.float32) for _ in range(b)]
        + [
            pltpu.SemaphoreType.DMA,
            pltpu.SemaphoreType.DMA,
            pltpu.SemaphoreType.DMA,
        ],
    )
    def emb_kernel(idx_hbm, table_hbm, pe_hbm, out_hbm, idx_v, pe_v, *rest):
        rows = rest[:b]
        gsem, psem, osem = rest[b:]
        wid = lax.axis_index("s") * nc + lax.axis_index("c")
        pos0 = wid * blk

        pe_cp = pltpu.async_copy(pe_hbm.at[pl.ds(pos0, blk)], pe_v, psem)
        pltpu.sync_copy(idx_hbm.at[:, pl.ds(pos0, blk)], idx_v)
        gathers = []
        for r in range(nsub):
            for j in range(b):
                gathers.append(
                    pltpu.async_copy(
                        table_hbm.at[idx_v.at[j, pl.ds(r * sub, sub)]],
                        rows[j].at[pl.ds(r * sub, sub)],
                        gsem,
                    )
                )
        pe_cp.wait()

        stores = []
        for r in range(nsub):
            for j in range(b):
                gathers[r * b + j].wait()

            def body(i, _):
                for k in range(d // lanes):
                    sl = pl.ds(k * lanes, lanes)
                    pv = pe_v[i, sl]
                    for j in range(b):
                        rows[j][i, sl] = rows[j][i, sl] * scale + pv
                return 0

            lax.fori_loop(r * sub, (r + 1) * sub, body, 0)
            for j in range(b):
                stores.append(
                    pltpu.async_copy(
                        rows[j].at[pl.ds(r * sub, sub)],
                        out_hbm.at[pl.ds(j * s + pos0 + r * sub, sub)],
                        osem,
                    )
                )
        for st in stores:
            st.wait()

    out = emb_kernel(x, table, pe)
    return out.reshape(b, s, d)
